# Initial kernel scaffold; baseline (speedup 1.0000x reference)
import jax
import jax.numpy as jnp
from jax.experimental import pallas as pl


def kernel(x, edge_index, batch, W1, b1, W2, b2, W3, b3, W4, b4, Wl, bl):
    raise NotImplementedError("write your pallas kernel here")


# speedup vs baseline: 1.0036x; 1.0036x over previous
import jax
import jax.numpy as jnp
from jax.experimental import pallas as pl


def _final_kernel(pooled_ref, wl_ref, bl_ref, out_ref):
    out_ref[...] = pooled_ref[...] @ wl_ref[...] + bl_ref[...]


def kernel(x, edge_index, batch, W1, b1, W2, b2, W3, b3, W4, b4, Wl, bl):
    def conv(h, Wa, ba, Wb, bb):
        src = edge_index[0]
        dst = edge_index[1]
        msgs = jnp.take(h, src, axis=0)
        agg = jnp.zeros_like(h).at[dst].add(msgs)
        t = h + agg
        t = jnp.maximum(t @ Wa + ba, 0.0)
        return t @ Wb + bb

    h = jnp.maximum(conv(x, W1, b1, W2, b2), 0.0)
    h = jnp.maximum(conv(h, W3, b3, W4, b4), 0.0)
    sums = jax.ops.segment_sum(h, batch, num_segments=256)
    counts = jax.ops.segment_sum(jnp.ones((h.shape[0],), h.dtype), batch, num_segments=256)
    pooled = sums / jnp.maximum(counts, 1.0)[:, None]
    return pl.pallas_call(
        _final_kernel,
        out_shape=jax.ShapeDtypeStruct((256, 32), jnp.float32),
    )(pooled, Wl, bl.reshape(1, 32))


# speedup vs baseline: 13.9150x; 13.8651x over previous
import jax
import jax.numpy as jnp
from jax import lax
from jax.experimental import pallas as pl
from jax.experimental.pallas import tpu as pltpu
from jax.experimental.pallas import tpu_sc as plsc

N = 100000
E = 6400000
NG = 256
HID = 64

S_PAD = 51200
E_PAD = S_PAD * 128
ACC_ROWS = 100352
ZROWS = 392
WB = 6272
WB_LAST = N - 15 * WB

BN = 2000
GRID = N // BN


def _zero_acc(sub, zbuf, acc):
    def zb(i, _):
        zbuf[i, :] = jnp.zeros((16,), jnp.float32)
        return 0
    lax.fori_loop(0, ZROWS, zb, 0)
    for z in range(16):
        pltpu.sync_copy(zbuf, acc.at[pl.ds((sub * 16 + z) * ZROWS, ZROWS)])


def _writeback(sub, acc, out_slice_fn):
    row0 = sub * WB

    @pl.when(sub < 15)
    def _():
        pltpu.sync_copy(acc.at[pl.ds(row0, WB)], out_slice_fn(row0, WB))

    @pl.when(sub == 15)
    def _():
        pltpu.sync_copy(acc.at[pl.ds(row0, WB_LAST)], out_slice_fn(row0, WB_LAST))


def _edge_sweep(table, src2d, dst2d, acc, sidx, didx, rows,
                isems, gsem, ssems, base, nsup):
    nb = nsup // 4

    def idx_issue(slot, sb):
        pltpu.async_copy(src2d.at[pl.ds(sb, 4)], sidx.at[slot], isems[slot])
        pltpu.async_copy(dst2d.at[pl.ds(sb, 4)], didx.at[slot], isems[slot])

    def idx_wait(slot):
        pltpu.make_async_copy(
            src2d.at[pl.ds(0, 4)], sidx.at[slot], isems[slot]).wait()
        pltpu.make_async_copy(
            dst2d.at[pl.ds(0, 4)], didx.at[slot], isems[slot]).wait()

    def scat_drain(r):
        for j in range(4):
            pltpu.make_async_copy(
                rows.at[r, j], acc.at[pl.ds(0, 128)], ssems[r]).wait()

    idx_issue(0, base)
    idx_issue(1, base + 4)

    def body(i, _):
        for q in range(4):
            r = q % 2
            s = i * 4 + q
            if q < 2:
                @pl.when(i > 0)
                def _():
                    scat_drain(r)
            else:
                scat_drain(r)
            tgt = (q + 2) % 4
            if q < 2:
                idx_issue(tgt, base + (s + 2) * 4)
            else:
                @pl.when(i < nb - 1)
                def _():
                    idx_issue(tgt, base + (s + 2) * 4)
            idx_wait(q)
            gcps = [pltpu.async_copy(table.at[sidx.at[q, j]], rows.at[r, j],
                                     gsem) for j in range(4)]
            for c in gcps:
                c.wait()
            for j in range(4):
                pltpu.async_copy(rows.at[r, j], acc.at[didx.at[q, j]],
                                 ssems[r], add=True)
        return 0

    lax.fori_loop(0, nb, body, 0)
    scat_drain(0)
    scat_drain(1)


def _sc_agg1_body(xp, src2d, dst2d, parts, acc, zbuf, sidx, didx,
                  isem0, isem1, isem2, isem3, gsem, ssem0, ssem1):
    core = lax.axis_index("c")
    sub = lax.axis_index("s")
    w = core * 16 + sub
    isems = [isem0, isem1, isem2, isem3]
    ssems = [ssem0, ssem1]

    def inner(rows):
        _zero_acc(sub, zbuf, acc)
        plsc.subcore_barrier()
        _edge_sweep(xp, src2d, dst2d, acc, sidx, didx, rows, isems, gsem,
                    ssems, base=w * (S_PAD // 32), nsup=S_PAD // 32 // 4)
        plsc.subcore_barrier()
        _writeback(sub, acc, lambda r0, n: parts.at[core, pl.ds(r0, n)])

    pl.run_scoped(inner, pltpu.VMEM((2, 4, 128, 16), jnp.float32))


def _sc_agg2_body(h1c3, src2d, dst2d, agg, acc, zbuf, sidx, didx,
                  isem0, isem1, isem2, isem3, gsem, ssem0, ssem1):
    core = lax.axis_index("c")
    sub = lax.axis_index("s")
    isems = [isem0, isem1, isem2, isem3]
    ssems = [ssem0, ssem1]

    def inner(rows):
        for r in range(2):
            chunk = core * 2 + r
            _zero_acc(sub, zbuf, acc)
            plsc.subcore_barrier()
            _edge_sweep(h1c3.at[chunk], src2d, dst2d, acc, sidx, didx, rows,
                        isems, gsem, ssems,
                        base=sub * (S_PAD // 16), nsup=S_PAD // 16 // 4)
            plsc.subcore_barrier()
            _writeback(sub, acc, lambda r0, n: agg.at[chunk, pl.ds(r0, n)])
            plsc.subcore_barrier()

    pl.run_scoped(inner, pltpu.VMEM((2, 4, 128, 16), jnp.float32))


_SC_SCRATCH = [
    pltpu.VMEM_SHARED((ACC_ROWS, 16), jnp.float32),
    pltpu.VMEM((ZROWS, 16), jnp.float32),
    pltpu.VMEM((4, 4, 128), jnp.int32),
    pltpu.VMEM((4, 4, 128), jnp.int32),
] + [pltpu.SemaphoreType.DMA] * 7


def _make_sc_agg1():
    mesh = plsc.VectorSubcoreMesh(core_axis_name="c", subcore_axis_name="s")
    return pl.kernel(
        _sc_agg1_body,
        out_type=jax.ShapeDtypeStruct((2, N, 16), jnp.float32),
        mesh=mesh,
        compiler_params=pltpu.CompilerParams(use_tc_tiling_on_sc=False),
        scratch_types=list(_SC_SCRATCH),
    )


def _make_sc_agg2():
    mesh = plsc.VectorSubcoreMesh(core_axis_name="c", subcore_axis_name="s")
    return pl.kernel(
        _sc_agg2_body,
        out_type=jax.ShapeDtypeStruct((4, N, 16), jnp.float32),
        mesh=mesh,
        compiler_params=pltpu.CompilerParams(use_tc_tiling_on_sc=False),
        scratch_types=list(_SC_SCRATCH),
    )


def _mlp1_kernel(xp_ref, parts_ref, w1_ref, b1_ref, w2_ref, b2_ref, out_ref):
    s = xp_ref[...] + parts_ref[0] + parts_ref[1]
    h = jnp.maximum(
        jnp.dot(s, w1_ref[...], preferred_element_type=jnp.float32)
        + b1_ref[...], 0.0)
    h1 = jnp.maximum(
        jnp.dot(h, w2_ref[...], preferred_element_type=jnp.float32)
        + b2_ref[...], 0.0)
    for c in range(4):
        out_ref[c] = h1[:, c * 16:(c + 1) * 16]


def _final_kernel(h1c_ref, agg_ref, batch_ref, w3_ref, b3_ref, w4_ref, b4_ref,
                  wl_ref, bl_ref, out_ref, acc_ref):
    i = pl.program_id(0)
    t = jnp.zeros((BN, HID), jnp.float32)
    for c in range(4):
        t = t + jnp.dot(h1c_ref[c] + agg_ref[c],
                        w3_ref[c * 16:(c + 1) * 16, :],
                        preferred_element_type=jnp.float32)
    h = jnp.maximum(t + b3_ref[...], 0.0)
    h2 = jnp.maximum(
        jnp.dot(h, w4_ref[...], preferred_element_type=jnp.float32)
        + b4_ref[...], 0.0)
    bb = batch_ref[0, 0]
    oh = (bb[:, None] == lax.broadcasted_iota(jnp.int32, (1, NG), 1)
          ).astype(jnp.float32)
    ext = jnp.concatenate([h2, jnp.ones((BN, HID), jnp.float32)], axis=1)
    contrib = lax.dot_general(oh, ext, (((0,), (0,)), ((), ())),
                              preferred_element_type=jnp.float32)

    @pl.when(i == 0)
    def _():
        acc_ref[...] = contrib

    @pl.when(i > 0)
    def _():
        acc_ref[...] = acc_ref[...] + contrib

    @pl.when(i == GRID - 1)
    def _():
        sums = acc_ref[:, :HID]
        cnts = acc_ref[:, HID:]
        pooled = sums / jnp.maximum(cnts, 1.0)
        out_ref[...] = (
            jnp.dot(pooled, wl_ref[...], preferred_element_type=jnp.float32)
            + bl_ref[...])


def kernel(x, edge_index, batch, W1, b1, W2, b2, W3, b3, W4, b4, Wl, bl):
    f32 = jnp.float32
    xp = jnp.pad(x, ((0, 0), (0, 16 - x.shape[1])))
    pad_e = E_PAD - E
    srcp = jnp.concatenate(
        [edge_index[0], jnp.zeros((pad_e,), jnp.int32)]).reshape(S_PAD, 128)
    dstp = jnp.concatenate(
        [edge_index[1], jnp.full((pad_e,), N, jnp.int32)]).reshape(S_PAD, 128)
    W1p = jnp.pad(W1, ((0, 16 - W1.shape[0]), (0, 0)))
    b1r = b1.reshape(1, HID)
    b2r = b2.reshape(1, HID)
    b3r = b3.reshape(1, HID)
    b4r = b4.reshape(1, HID)
    blr = bl.reshape(1, 32)
    batch2d = batch.reshape(GRID, 1, BN)

    parts = _make_sc_agg1()(xp, srcp, dstp)

    h1c = pl.pallas_call(
        _mlp1_kernel,
        grid=(GRID,),
        in_specs=[
            pl.BlockSpec((BN, 16), lambda i: (i, 0)),
            pl.BlockSpec((2, BN, 16), lambda i: (0, i, 0)),
            pl.BlockSpec((16, HID), lambda i: (0, 0)),
            pl.BlockSpec((1, HID), lambda i: (0, 0)),
            pl.BlockSpec((HID, HID), lambda i: (0, 0)),
            pl.BlockSpec((1, HID), lambda i: (0, 0)),
        ],
        out_specs=pl.BlockSpec((4, BN, 16), lambda i: (0, i, 0)),
        out_shape=jax.ShapeDtypeStruct((4, N, 16), f32),
    )(xp, parts, W1p, b1r, W2, b2r)

    agg = _make_sc_agg2()(h1c, srcp, dstp)

    out = pl.pallas_call(
        _final_kernel,
        grid=(GRID,),
        in_specs=[
            pl.BlockSpec((4, BN, 16), lambda i: (0, i, 0)),
            pl.BlockSpec((4, BN, 16), lambda i: (0, i, 0)),
            pl.BlockSpec((1, 1, BN), lambda i: (i, 0, 0)),
            pl.BlockSpec((HID, HID), lambda i: (0, 0)),
            pl.BlockSpec((1, HID), lambda i: (0, 0)),
            pl.BlockSpec((HID, HID), lambda i: (0, 0)),
            pl.BlockSpec((1, HID), lambda i: (0, 0)),
            pl.BlockSpec((HID, 32), lambda i: (0, 0)),
            pl.BlockSpec((1, 32), lambda i: (0, 0)),
        ],
        out_specs=pl.BlockSpec((NG, 32), lambda i: (0, 0)),
        out_shape=jax.ShapeDtypeStruct((NG, 32), f32),
        scratch_shapes=[pltpu.VMEM((NG, 128), f32)],
    )(h1c, agg, batch2d, W3, b3r, W4, b4r, Wl, blr)
    return out


# speedup vs baseline: 33.9873x; 2.4425x over previous
import jax
import jax.numpy as jnp
from jax import lax
from jax.experimental import pallas as pl
from jax.experimental.pallas import tpu as pltpu
from jax.experimental.pallas import tpu_sc as plsc

N = 100000
E = 6400000
NG = 256
HID = 64

S_PAD = 50176
E_PAD = S_PAD * 128
ACC_ROWS = 100352
ZROWS = 392
WB = 6272
WB_LAST = N - 15 * WB

BN = 2000
GRID = N // BN


def _zero_acc(sub, zbuf, acc):
    zero_vec = jnp.zeros(zbuf.shape[1:], zbuf.dtype)

    def zb(i, _):
        zbuf[i, :] = zero_vec
        return 0
    lax.fori_loop(0, ZROWS, zb, 0)
    for z in range(16):
        pltpu.sync_copy(zbuf, acc.at[pl.ds((sub * 16 + z) * ZROWS, ZROWS)])


def _writeback(sub, acc, out_slice_fn):
    row0 = sub * WB

    @pl.when(sub < 15)
    def _():
        pltpu.sync_copy(acc.at[pl.ds(row0, WB)], out_slice_fn(row0, WB))

    @pl.when(sub == 15)
    def _():
        pltpu.sync_copy(acc.at[pl.ds(row0, WB_LAST)], out_slice_fn(row0, WB_LAST))


def _edge_sweep(table, src2d, dst2d, acc, sidx, didx, rows,
                isems, gsem, ssems, base, nsup):
    nb = nsup // 4

    def idx_issue(slot, sb):
        pltpu.async_copy(src2d.at[pl.ds(sb, 4)], sidx.at[slot], isems[slot])
        pltpu.async_copy(dst2d.at[pl.ds(sb, 4)], didx.at[slot], isems[slot])

    def idx_wait(slot):
        pltpu.make_async_copy(
            src2d.at[pl.ds(0, 4)], sidx.at[slot], isems[slot]).wait()
        pltpu.make_async_copy(
            dst2d.at[pl.ds(0, 4)], didx.at[slot], isems[slot]).wait()

    def scat_drain(r):
        for j in range(4):
            pltpu.make_async_copy(
                rows.at[r, j], acc.at[pl.ds(0, 128)], ssems[r]).wait()

    idx_issue(0, base)
    idx_issue(1, base + 4)

    def body(i, _):
        for q in range(4):
            r = q % 2
            s = i * 4 + q
            if q < 2:
                @pl.when(i > 0)
                def _():
                    scat_drain(r)
            else:
                scat_drain(r)
            tgt = (q + 2) % 4
            if q < 2:
                idx_issue(tgt, base + (s + 2) * 4)
            else:
                @pl.when(i < nb - 1)
                def _():
                    idx_issue(tgt, base + (s + 2) * 4)
            idx_wait(q)
            gcps = [pltpu.async_copy(table.at[sidx.at[q, j]], rows.at[r, j],
                                     gsem) for j in range(4)]
            for c in gcps:
                c.wait()
            for j in range(4):
                pltpu.async_copy(rows.at[r, j], acc.at[didx.at[q, j]],
                                 ssems[r], add=True)
        return 0

    lax.fori_loop(0, nb, body, 0)
    scat_drain(0)
    scat_drain(1)


def _sc_agg1_body(xp, src2d, dst2d, parts, acc, zbuf, sidx, didx,
                  isem0, isem1, isem2, isem3, gsem, ssem0, ssem1):
    core = lax.axis_index("c")
    sub = lax.axis_index("s")
    w = core * 16 + sub
    isems = [isem0, isem1, isem2, isem3]
    ssems = [ssem0, ssem1]

    def inner(rows):
        _zero_acc(sub, zbuf, acc)
        plsc.subcore_barrier()
        _edge_sweep(xp, src2d, dst2d, acc, sidx, didx, rows, isems, gsem,
                    ssems, base=w * (S_PAD // 32), nsup=S_PAD // 32 // 4)
        plsc.subcore_barrier()
        _writeback(sub, acc, lambda r0, n: parts.at[core, pl.ds(r0, n)])

    pl.run_scoped(inner, pltpu.VMEM((2, 4, 128, 16), jnp.float32))


def _sc_agg2_body(h1h, src2d, dst2d, agg, acc, zbuf, sidx, didx,
                  isem0, isem1, isem2, isem3, gsem, ssem0, ssem1):
    core = lax.axis_index("c")
    sub = lax.axis_index("s")
    isems = [isem0, isem1, isem2, isem3]
    ssems = [ssem0, ssem1]

    def inner(rows):
        _zero_acc(sub, zbuf, acc)
        plsc.subcore_barrier()
        _edge_sweep(h1h.at[core], src2d, dst2d, acc, sidx, didx, rows,
                    isems, gsem, ssems,
                    base=sub * (S_PAD // 16), nsup=S_PAD // 16 // 4)
        plsc.subcore_barrier()
        _writeback(sub, acc, lambda r0, n: agg.at[core, pl.ds(r0, n)])

    pl.run_scoped(inner, pltpu.VMEM((2, 4, 128, 32), jnp.bfloat16))


_SC_SCRATCH1 = [
    pltpu.VMEM_SHARED((ACC_ROWS, 16), jnp.float32),
    pltpu.VMEM((ZROWS, 16), jnp.float32),
    pltpu.VMEM((4, 4, 128), jnp.int32),
    pltpu.VMEM((4, 4, 128), jnp.int32),
] + [pltpu.SemaphoreType.DMA] * 7

_SC_SCRATCH2 = [
    pltpu.VMEM_SHARED((ACC_ROWS, 32), jnp.bfloat16),
    pltpu.VMEM((ZROWS, 32), jnp.bfloat16),
    pltpu.VMEM((4, 4, 128), jnp.int32),
    pltpu.VMEM((4, 4, 128), jnp.int32),
] + [pltpu.SemaphoreType.DMA] * 7


def _make_sc_agg1():
    mesh = plsc.VectorSubcoreMesh(core_axis_name="c", subcore_axis_name="s")
    return pl.kernel(
        _sc_agg1_body,
        out_type=jax.ShapeDtypeStruct((2, N, 16), jnp.float32),
        mesh=mesh,
        compiler_params=pltpu.CompilerParams(use_tc_tiling_on_sc=False),
        scratch_types=list(_SC_SCRATCH1),
    )


def _make_sc_agg2():
    mesh = plsc.VectorSubcoreMesh(core_axis_name="c", subcore_axis_name="s")
    return pl.kernel(
        _sc_agg2_body,
        out_type=jax.ShapeDtypeStruct((2, N, 32), jnp.bfloat16),
        mesh=mesh,
        compiler_params=pltpu.CompilerParams(use_tc_tiling_on_sc=False),
        scratch_types=list(_SC_SCRATCH2),
    )


def _mlp1_kernel(xp_ref, parts_ref, w1_ref, b1_ref, w2_ref, b2_ref, out_ref,
                 outh_ref):
    s = xp_ref[...] + parts_ref[0] + parts_ref[1]
    h = jnp.maximum(
        jnp.dot(s, w1_ref[...], preferred_element_type=jnp.float32)
        + b1_ref[...], 0.0)
    h1 = jnp.maximum(
        jnp.dot(h, w2_ref[...], preferred_element_type=jnp.float32)
        + b2_ref[...], 0.0)
    for c in range(4):
        out_ref[c] = h1[:, c * 16:(c + 1) * 16]
    for hh in range(2):
        outh_ref[hh] = h1[:, hh * 32:(hh + 1) * 32].astype(jnp.bfloat16)


def _final_kernel(h1c_ref, agg_ref, batch_ref, w3_ref, b3_ref, w4_ref, b4_ref,
                  wl_ref, bl_ref, out_ref, acc_ref):
    i = pl.program_id(0)
    t = jnp.zeros((BN, HID), jnp.float32)
    for c in range(4):
        t = t + jnp.dot(h1c_ref[c], w3_ref[c * 16:(c + 1) * 16, :],
                        preferred_element_type=jnp.float32)
    for hh in range(2):
        t = t + jnp.dot(agg_ref[hh].astype(jnp.float32),
                        w3_ref[hh * 32:(hh + 1) * 32, :],
                        preferred_element_type=jnp.float32)
    h = jnp.maximum(t + b3_ref[...], 0.0)
    h2 = jnp.maximum(
        jnp.dot(h, w4_ref[...], preferred_element_type=jnp.float32)
        + b4_ref[...], 0.0)
    bb = batch_ref[0, 0]
    oh = (bb[:, None] == lax.broadcasted_iota(jnp.int32, (1, NG), 1)
          ).astype(jnp.float32)
    ext = jnp.concatenate([h2, jnp.ones((BN, HID), jnp.float32)], axis=1)
    contrib = lax.dot_general(oh, ext, (((0,), (0,)), ((), ())),
                              preferred_element_type=jnp.float32)

    @pl.when(i == 0)
    def _():
        acc_ref[...] = contrib

    @pl.when(i > 0)
    def _():
        acc_ref[...] = acc_ref[...] + contrib

    @pl.when(i == GRID - 1)
    def _():
        sums = acc_ref[:, :HID]
        cnts = acc_ref[:, HID:]
        pooled = sums / jnp.maximum(cnts, 1.0)
        out_ref[...] = (
            jnp.dot(pooled, wl_ref[...], preferred_element_type=jnp.float32)
            + bl_ref[...])


def kernel(x, edge_index, batch, W1, b1, W2, b2, W3, b3, W4, b4, Wl, bl):
    f32 = jnp.float32
    xp = jnp.pad(x, ((0, 0), (0, 16 - x.shape[1])))
    pad_e = E_PAD - E
    srcp = jnp.concatenate(
        [edge_index[0], jnp.zeros((pad_e,), jnp.int32)]).reshape(S_PAD, 128)
    dstp = jnp.concatenate(
        [edge_index[1], jnp.full((pad_e,), N, jnp.int32)]).reshape(S_PAD, 128)
    W1p = jnp.pad(W1, ((0, 16 - W1.shape[0]), (0, 0)))
    b1r = b1.reshape(1, HID)
    b2r = b2.reshape(1, HID)
    b3r = b3.reshape(1, HID)
    b4r = b4.reshape(1, HID)
    blr = bl.reshape(1, 32)
    batch2d = batch.reshape(GRID, 1, BN)

    parts = _make_sc_agg1()(xp, srcp, dstp)

    _mlp1_out = pl.pallas_call(
        _mlp1_kernel,
        grid=(GRID,),
        in_specs=[
            pl.BlockSpec((BN, 16), lambda i: (i, 0)),
            pl.BlockSpec((2, BN, 16), lambda i: (0, i, 0)),
            pl.BlockSpec((16, HID), lambda i: (0, 0)),
            pl.BlockSpec((1, HID), lambda i: (0, 0)),
            pl.BlockSpec((HID, HID), lambda i: (0, 0)),
            pl.BlockSpec((1, HID), lambda i: (0, 0)),
        ],
        out_specs=[pl.BlockSpec((4, BN, 16), lambda i: (0, i, 0)),
                   pl.BlockSpec((2, BN, 32), lambda i: (0, i, 0))],
        out_shape=[jax.ShapeDtypeStruct((4, N, 16), f32),
                   jax.ShapeDtypeStruct((2, N, 32), jnp.bfloat16)],
    )(xp, parts, W1p, b1r, W2, b2r)
    h1c, h1h = _mlp1_out

    agg = _make_sc_agg2()(h1h, srcp, dstp)

    out = pl.pallas_call(
        _final_kernel,
        grid=(GRID,),
        in_specs=[
            pl.BlockSpec((4, BN, 16), lambda i: (0, i, 0)),
            pl.BlockSpec((2, BN, 32), lambda i: (0, i, 0)),
            pl.BlockSpec((1, 1, BN), lambda i: (i, 0, 0)),
            pl.BlockSpec((HID, HID), lambda i: (0, 0)),
            pl.BlockSpec((1, HID), lambda i: (0, 0)),
            pl.BlockSpec((HID, HID), lambda i: (0, 0)),
            pl.BlockSpec((1, HID), lambda i: (0, 0)),
            pl.BlockSpec((HID, 32), lambda i: (0, 0)),
            pl.BlockSpec((1, 32), lambda i: (0, 0)),
        ],
        out_specs=pl.BlockSpec((NG, 32), lambda i: (0, 0)),
        out_shape=jax.ShapeDtypeStruct((NG, 32), f32),
        scratch_shapes=[pltpu.VMEM((NG, 128), f32)],
    )(h1c, agg, batch2d, W3, b3r, W4, b4r, Wl, blr)
    return out


# speedup vs baseline: 35.7974x; 1.0533x over previous
import jax
import jax.numpy as jnp
from jax import lax
from jax.experimental import pallas as pl
from jax.experimental.pallas import tpu as pltpu
from jax.experimental.pallas import tpu_sc as plsc

N = 100000
E = 6400000
NG = 256
HID = 64

S_PAD = 50176
E_PAD = S_PAD * 128
ACC_ROWS = 100352
ZROWS = 392
WB = 6272
DUMP = 100351

NP = 102400
BN = 2048
GRID = NP // BN


def _sigma(a):
    o = a % 2048
    return (a - o) + 8 * (o % 256) + o // 256


def _zero_acc(sub, zbuf, acc):
    zero_vec = jnp.zeros(zbuf.shape[1:], zbuf.dtype)

    def zb(i, _):
        zbuf[i, :] = zero_vec
        return 0
    lax.fori_loop(0, ZROWS, zb, 0)
    for z in range(16):
        pltpu.sync_copy(zbuf, acc.at[pl.ds((sub * 16 + z) * ZROWS, ZROWS)])


def _writeback(sub, acc, zbuf, out_slice_fn):
    row0 = sub * WB
    pltpu.sync_copy(acc.at[pl.ds(row0, WB)], out_slice_fn(row0, WB))

    @pl.when(sub == 15)
    def _():
        for k in range(5):
            pltpu.sync_copy(zbuf, out_slice_fn(ACC_ROWS + k * ZROWS, ZROWS))
        rem = NP - ACC_ROWS - 5 * ZROWS
        pltpu.sync_copy(zbuf.at[pl.ds(0, rem)],
                        out_slice_fn(ACC_ROWS + 5 * ZROWS, rem))


def _edge_sweep(table, src2d, dst2d, acc, sidx, didx, rows,
                isems, gsem, ssems, base, nsup):
    nb = nsup // 4

    def idx_issue(slot, sb):
        pltpu.async_copy(src2d.at[pl.ds(sb, 4)], sidx.at[slot], isems[slot])
        pltpu.async_copy(dst2d.at[pl.ds(sb, 4)], didx.at[slot], isems[slot])

    def idx_wait(slot):
        pltpu.make_async_copy(
            src2d.at[pl.ds(0, 4)], sidx.at[slot], isems[slot]).wait()
        pltpu.make_async_copy(
            dst2d.at[pl.ds(0, 4)], didx.at[slot], isems[slot]).wait()

    def scat_drain(r):
        for j in range(4):
            pltpu.make_async_copy(
                rows.at[r, j], acc.at[pl.ds(0, 128)], ssems[r]).wait()

    idx_issue(0, base)
    idx_issue(1, base + 4)

    def body(i, _):
        for q in range(4):
            r = q % 2
            s = i * 4 + q
            if q < 2:
                @pl.when(i > 0)
                def _():
                    scat_drain(r)
            else:
                scat_drain(r)
            tgt = (q + 2) % 4
            if q < 2:
                idx_issue(tgt, base + (s + 2) * 4)
            else:
                @pl.when(i < nb - 1)
                def _():
                    idx_issue(tgt, base + (s + 2) * 4)
            idx_wait(q)
            gcps = [pltpu.async_copy(table.at[sidx.at[q, j]], rows.at[r, j],
                                     gsem) for j in range(4)]
            for c in gcps:
                c.wait()
            for j in range(4):
                pltpu.async_copy(rows.at[r, j], acc.at[didx.at[q, j]],
                                 ssems[r], add=True)
        return 0

    lax.fori_loop(0, nb, body, 0)
    scat_drain(0)
    scat_drain(1)


def _sc_agg1_body(xp, src2d, dst2d, parts, acc, zbuf, sidx, didx,
                  isem0, isem1, isem2, isem3, gsem, ssem0, ssem1):
    core = lax.axis_index("c")
    sub = lax.axis_index("s")
    w = core * 16 + sub
    isems = [isem0, isem1, isem2, isem3]
    ssems = [ssem0, ssem1]

    def inner(rows):
        _zero_acc(sub, zbuf, acc)
        plsc.subcore_barrier()
        _edge_sweep(xp, src2d, dst2d, acc, sidx, didx, rows, isems, gsem,
                    ssems, base=w * (S_PAD // 32), nsup=S_PAD // 32 // 4)
        plsc.subcore_barrier()
        _writeback(sub, acc, zbuf,
                   lambda r0, n: parts.at[core, pl.ds(r0, n)])

    pl.run_scoped(inner, pltpu.VMEM((2, 4, 128, 16), jnp.float32))


def _sc_agg2_body(h1h, src2d, dst2d, agg, acc, zbuf, sidx, didx,
                  isem0, isem1, isem2, isem3, gsem, ssem0, ssem1):
    core = lax.axis_index("c")
    sub = lax.axis_index("s")
    isems = [isem0, isem1, isem2, isem3]
    ssems = [ssem0, ssem1]

    def inner(rows):
        _zero_acc(sub, zbuf, acc)
        plsc.subcore_barrier()
        _edge_sweep(h1h.at[core], src2d, dst2d, acc, sidx, didx, rows,
                    isems, gsem, ssems,
                    base=sub * (S_PAD // 16), nsup=S_PAD // 16 // 4)
        plsc.subcore_barrier()
        _writeback(sub, acc, zbuf,
                   lambda r0, n: agg.at[core, pl.ds(r0, n)])

    pl.run_scoped(inner, pltpu.VMEM((2, 4, 128, 32), jnp.bfloat16))


_SC_SCRATCH1 = [
    pltpu.VMEM_SHARED((ACC_ROWS, 16), jnp.float32),
    pltpu.VMEM((ZROWS, 16), jnp.float32),
    pltpu.VMEM((4, 4, 128), jnp.int32),
    pltpu.VMEM((4, 4, 128), jnp.int32),
] + [pltpu.SemaphoreType.DMA] * 7

_SC_SCRATCH2 = [
    pltpu.VMEM_SHARED((ACC_ROWS, 32), jnp.bfloat16),
    pltpu.VMEM((ZROWS, 32), jnp.bfloat16),
    pltpu.VMEM((4, 4, 128), jnp.int32),
    pltpu.VMEM((4, 4, 128), jnp.int32),
] + [pltpu.SemaphoreType.DMA] * 7


def _make_sc_agg1():
    mesh = plsc.VectorSubcoreMesh(core_axis_name="c", subcore_axis_name="s")
    return pl.kernel(
        _sc_agg1_body,
        out_type=jax.ShapeDtypeStruct((2, NP, 16), jnp.float32),
        mesh=mesh,
        compiler_params=pltpu.CompilerParams(use_tc_tiling_on_sc=False),
        scratch_types=list(_SC_SCRATCH1),
    )


def _make_sc_agg2():
    mesh = plsc.VectorSubcoreMesh(core_axis_name="c", subcore_axis_name="s")
    return pl.kernel(
        _sc_agg2_body,
        out_type=jax.ShapeDtypeStruct((2, NP, 32), jnp.bfloat16),
        mesh=mesh,
        compiler_params=pltpu.CompilerParams(use_tc_tiling_on_sc=False),
        scratch_types=list(_SC_SCRATCH2),
    )


def _unpack8(blk, w):
    return jnp.concatenate([blk[:, g * w:(g + 1) * w] for g in range(8)],
                           axis=0)


def _pack8(nat, w):
    return jnp.concatenate([nat[g * 256:(g + 1) * 256, :] for g in range(8)],
                           axis=1)


def _mlp1_kernel(x_ref, parts_ref, w1_ref, w1p_ref, b1_ref, w2_ref, b2_ref,
                 out_ref, outh_ref):
    pp = _unpack8(parts_ref[0], 16) + _unpack8(parts_ref[1], 16)
    t = (jnp.dot(x_ref[...], w1_ref[...], preferred_element_type=jnp.float32)
         + jnp.dot(pp, w1p_ref[...], preferred_element_type=jnp.float32)
         + b1_ref[...])
    h = jnp.maximum(t, 0.0)
    h1 = jnp.maximum(
        jnp.dot(h, w2_ref[...], preferred_element_type=jnp.float32)
        + b2_ref[...], 0.0)
    for c in range(4):
        out_ref[c] = h1[:, c * 16:(c + 1) * 16]
    for hh in range(2):
        outh_ref[hh] = _pack8(
            h1[:, hh * 32:(hh + 1) * 32].astype(jnp.bfloat16), 32)


def _final_kernel(h1c_ref, agg_ref, batch_ref, w3_ref, b3_ref, w4_ref, b4_ref,
                  wl_ref, bl_ref, out_ref, acc_ref):
    i = pl.program_id(0)
    t = jnp.zeros((BN, HID), jnp.float32)
    for c in range(4):
        t = t + jnp.dot(h1c_ref[c], w3_ref[c * 16:(c + 1) * 16, :],
                        preferred_element_type=jnp.float32)
    for hh in range(2):
        t = t + jnp.dot(_unpack8(agg_ref[hh], 32).astype(jnp.float32),
                        w3_ref[hh * 32:(hh + 1) * 32, :],
                        preferred_element_type=jnp.float32)
    h = jnp.maximum(t + b3_ref[...], 0.0)
    h2 = jnp.maximum(
        jnp.dot(h, w4_ref[...], preferred_element_type=jnp.float32)
        + b4_ref[...], 0.0)
    bb = batch_ref[0, 0]
    oh = (bb[:, None] == lax.broadcasted_iota(jnp.int32, (1, NG), 1)
          ).astype(jnp.float32)
    ext = jnp.concatenate([h2, jnp.ones((BN, HID), jnp.float32)], axis=1)
    contrib = lax.dot_general(oh, ext, (((0,), (0,)), ((), ())),
                              preferred_element_type=jnp.float32)

    @pl.when(i == 0)
    def _():
        acc_ref[...] = contrib

    @pl.when(i > 0)
    def _():
        acc_ref[...] = acc_ref[...] + contrib

    @pl.when(i == GRID - 1)
    def _():
        sums = acc_ref[:, :HID]
        cnts = acc_ref[:, HID:]
        pooled = sums / jnp.maximum(cnts, 1.0)
        out_ref[...] = (
            jnp.dot(pooled, wl_ref[...], preferred_element_type=jnp.float32)
            + bl_ref[...])


def kernel(x, edge_index, batch, W1, b1, W2, b2, W3, b3, W4, b4, Wl, bl):
    f32 = jnp.float32
    xp = jnp.pad(x, ((0, NP - N), (0, 16 - x.shape[1])))
    xpad9 = jnp.pad(x, ((0, NP - N), (0, 0)))
    pad_e = E_PAD - E
    src = edge_index[0]
    srcp = jnp.concatenate(
        [src, jnp.zeros((pad_e,), jnp.int32)]).reshape(S_PAD, 128)
    srcp_s = jnp.concatenate(
        [_sigma(src), jnp.zeros((pad_e,), jnp.int32)]).reshape(S_PAD, 128)
    dstp_s = jnp.concatenate(
        [_sigma(edge_index[1]),
         jnp.full((pad_e,), DUMP, jnp.int32)]).reshape(S_PAD, 128)
    W1p = jnp.pad(W1, ((0, 16 - W1.shape[0]), (0, 0)))
    b1r = b1.reshape(1, HID)
    b2r = b2.reshape(1, HID)
    b3r = b3.reshape(1, HID)
    b4r = b4.reshape(1, HID)
    blr = bl.reshape(1, 32)
    batch2d = jnp.concatenate(
        [batch, jnp.full((NP - N,), NG, jnp.int32)]).reshape(GRID, 1, BN)

    parts = _make_sc_agg1()(xp, srcp, dstp_s)
    parts2d = parts.reshape(2, NP * 16 // 128, 128)

    _mlp1_out = pl.pallas_call(
        _mlp1_kernel,
        grid=(GRID,),
        in_specs=[
            pl.BlockSpec((BN, 9), lambda i: (i, 0)),
            pl.BlockSpec((2, BN // 8, 128), lambda i: (0, i, 0)),
            pl.BlockSpec((9, HID), lambda i: (0, 0)),
            pl.BlockSpec((16, HID), lambda i: (0, 0)),
            pl.BlockSpec((1, HID), lambda i: (0, 0)),
            pl.BlockSpec((HID, HID), lambda i: (0, 0)),
            pl.BlockSpec((1, HID), lambda i: (0, 0)),
        ],
        out_specs=[pl.BlockSpec((4, BN, 16), lambda i: (0, i, 0)),
                   pl.BlockSpec((2, BN // 8, 256), lambda i: (0, i, 0))],
        out_shape=[jax.ShapeDtypeStruct((4, NP, 16), f32),
                   jax.ShapeDtypeStruct((2, NP * 32 // 256, 256),
                                        jnp.bfloat16)],
    )(xpad9, parts2d, W1, W1p, b1r, W2, b2r)
    h1c, h1h2d = _mlp1_out
    h1h = h1h2d.reshape(2, NP, 32)

    agg = _make_sc_agg2()(h1h, srcp_s, dstp_s)
    agg2d = agg.reshape(2, NP * 32 // 256, 256)

    out = pl.pallas_call(
        _final_kernel,
        grid=(GRID,),
        in_specs=[
            pl.BlockSpec((4, BN, 16), lambda i: (0, i, 0)),
            pl.BlockSpec((2, BN // 8, 256), lambda i: (0, i, 0)),
            pl.BlockSpec((1, 1, BN), lambda i: (i, 0, 0)),
            pl.BlockSpec((HID, HID), lambda i: (0, 0)),
            pl.BlockSpec((1, HID), lambda i: (0, 0)),
            pl.BlockSpec((HID, HID), lambda i: (0, 0)),
            pl.BlockSpec((1, HID), lambda i: (0, 0)),
            pl.BlockSpec((HID, 32), lambda i: (0, 0)),
            pl.BlockSpec((1, 32), lambda i: (0, 0)),
        ],
        out_specs=pl.BlockSpec((NG, 32), lambda i: (0, 0)),
        out_shape=jax.ShapeDtypeStruct((NG, 32), f32),
        scratch_shapes=[pltpu.VMEM((NG, 128), f32)],
    )(h1c, agg2d, batch2d, W3, b3r, W4, b4r, Wl, blr)
    return out


# speedup vs baseline: 35.8013x; 1.0001x over previous
import jax
import jax.numpy as jnp
from jax import lax
from jax.experimental import pallas as pl
from jax.experimental.pallas import tpu as pltpu
from jax.experimental.pallas import tpu_sc as plsc

N = 100000
E = 6400000
NG = 256
HID = 64

S_PAD = 50176
E_PAD = S_PAD * 128
ACC_ROWS = 100352
ZROWS = 392
WB = 6272
DUMP = 100351

NP = 102400
BN = 2048
GRID = NP // BN


def _zero_acc(sub, zbuf, acc):
    zero_vec = jnp.zeros(zbuf.shape[1:], zbuf.dtype)

    def zb(i, _):
        zbuf[i, :] = zero_vec
        return 0
    lax.fori_loop(0, ZROWS, zb, 0)
    for z in range(16):
        pltpu.sync_copy(zbuf, acc.at[pl.ds((sub * 16 + z) * ZROWS, ZROWS)])


def _writeback(sub, acc, zbuf, out_slice_fn):
    row0 = sub * WB
    pltpu.sync_copy(acc.at[pl.ds(row0, WB)], out_slice_fn(row0, WB))

    @pl.when(sub == 15)
    def _():
        for k in range(5):
            pltpu.sync_copy(zbuf, out_slice_fn(ACC_ROWS + k * ZROWS, ZROWS))
        rem = NP - ACC_ROWS - 5 * ZROWS
        pltpu.sync_copy(zbuf.at[pl.ds(0, rem)],
                        out_slice_fn(ACC_ROWS + 5 * ZROWS, rem))


def _sigma_slices(ref, q):
    for j in range(4):
        for k in range(8):
            v = ref[q, j, pl.ds(k * 16, 16)]
            o = v & 2047
            ref[q, j, pl.ds(k * 16, 16)] = (
                (v - o) | ((o & 255) << 3) | (o >> 8))


def _edge_sweep(table, src2d, dst2d, acc, sidx, didx, rows,
                isems, gsem, ssems, base, nsup, sig_src, sig_dst):
    nb = nsup // 4

    def idx_issue(slot, sb):
        pltpu.async_copy(src2d.at[pl.ds(sb, 4)], sidx.at[slot], isems[slot])
        pltpu.async_copy(dst2d.at[pl.ds(sb, 4)], didx.at[slot], isems[slot])

    def idx_wait(slot):
        pltpu.make_async_copy(
            src2d.at[pl.ds(0, 4)], sidx.at[slot], isems[slot]).wait()
        pltpu.make_async_copy(
            dst2d.at[pl.ds(0, 4)], didx.at[slot], isems[slot]).wait()

    def scat_drain(r):
        for j in range(4):
            pltpu.make_async_copy(
                rows.at[r, j], acc.at[pl.ds(0, 128)], ssems[r]).wait()

    idx_issue(0, base)
    idx_issue(1, base + 4)

    def body(i, _):
        for q in range(4):
            r = q % 2
            s = i * 4 + q
            if q < 2:
                @pl.when(i > 0)
                def _():
                    scat_drain(r)
            else:
                scat_drain(r)
            tgt = (q + 2) % 4
            if q < 2:
                idx_issue(tgt, base + (s + 2) * 4)
            else:
                @pl.when(i < nb - 1)
                def _():
                    idx_issue(tgt, base + (s + 2) * 4)
            idx_wait(q)
            if sig_src:
                _sigma_slices(sidx, q)
            gcps = [pltpu.async_copy(table.at[sidx.at[q, j]], rows.at[r, j],
                                     gsem) for j in range(4)]
            if sig_dst:
                _sigma_slices(didx, q)
            for c in gcps:
                c.wait()
            for j in range(4):
                pltpu.async_copy(rows.at[r, j], acc.at[didx.at[q, j]],
                                 ssems[r], add=True)
        return 0

    lax.fori_loop(0, nb, body, 0)
    scat_drain(0)
    scat_drain(1)


def _sc_agg1_body(xp, src2d, dst2d, parts, acc, zbuf, sidx, didx,
                  isem0, isem1, isem2, isem3, gsem, ssem0, ssem1):
    core = lax.axis_index("c")
    sub = lax.axis_index("s")
    w = core * 16 + sub
    isems = [isem0, isem1, isem2, isem3]
    ssems = [ssem0, ssem1]

    def inner(rows):
        _zero_acc(sub, zbuf, acc)
        plsc.subcore_barrier()
        _edge_sweep(xp, src2d, dst2d, acc, sidx, didx, rows, isems, gsem,
                    ssems, base=w * (S_PAD // 32), nsup=S_PAD // 32 // 4,
                    sig_src=False, sig_dst=True)
        plsc.subcore_barrier()
        _writeback(sub, acc, zbuf,
                   lambda r0, n: parts.at[core, pl.ds(r0, n)])

    pl.run_scoped(inner, pltpu.VMEM((2, 4, 128, 16), jnp.float32))


def _sc_agg2_body(h1h, src2d, dst2d, agg, acc, zbuf, sidx, didx,
                  isem0, isem1, isem2, isem3, gsem, ssem0, ssem1):
    core = lax.axis_index("c")
    sub = lax.axis_index("s")
    isems = [isem0, isem1, isem2, isem3]
    ssems = [ssem0, ssem1]

    def inner(rows):
        _zero_acc(sub, zbuf, acc)
        plsc.subcore_barrier()
        _edge_sweep(h1h.at[core], src2d, dst2d, acc, sidx, didx, rows,
                    isems, gsem, ssems,
                    base=sub * (S_PAD // 16), nsup=S_PAD // 16 // 4,
                    sig_src=True, sig_dst=True)
        plsc.subcore_barrier()
        _writeback(sub, acc, zbuf,
                   lambda r0, n: agg.at[core, pl.ds(r0, n)])

    pl.run_scoped(inner, pltpu.VMEM((2, 4, 128, 32), jnp.bfloat16))


_SC_SCRATCH1 = [
    pltpu.VMEM_SHARED((ACC_ROWS, 16), jnp.float32),
    pltpu.VMEM((ZROWS, 16), jnp.float32),
    pltpu.VMEM((4, 4, 128), jnp.int32),
    pltpu.VMEM((4, 4, 128), jnp.int32),
] + [pltpu.SemaphoreType.DMA] * 7

_SC_SCRATCH2 = [
    pltpu.VMEM_SHARED((ACC_ROWS, 32), jnp.bfloat16),
    pltpu.VMEM((ZROWS, 32), jnp.bfloat16),
    pltpu.VMEM((4, 4, 128), jnp.int32),
    pltpu.VMEM((4, 4, 128), jnp.int32),
] + [pltpu.SemaphoreType.DMA] * 7


def _make_sc_agg1():
    mesh = plsc.VectorSubcoreMesh(core_axis_name="c", subcore_axis_name="s")
    return pl.kernel(
        _sc_agg1_body,
        out_type=jax.ShapeDtypeStruct((2, NP, 16), jnp.float32),
        mesh=mesh,
        compiler_params=pltpu.CompilerParams(use_tc_tiling_on_sc=False),
        scratch_types=list(_SC_SCRATCH1),
    )


def _make_sc_agg2():
    mesh = plsc.VectorSubcoreMesh(core_axis_name="c", subcore_axis_name="s")
    return pl.kernel(
        _sc_agg2_body,
        out_type=jax.ShapeDtypeStruct((2, NP, 32), jnp.bfloat16),
        mesh=mesh,
        compiler_params=pltpu.CompilerParams(use_tc_tiling_on_sc=False),
        scratch_types=list(_SC_SCRATCH2),
    )


def _unpack8(blk, w):
    return jnp.concatenate([blk[:, g * w:(g + 1) * w] for g in range(8)],
                           axis=0)


def _pack8(nat, w):
    return jnp.concatenate([nat[g * 256:(g + 1) * 256, :] for g in range(8)],
                           axis=1)


def _mlp1_kernel(x_ref, parts_ref, w1_ref, w1p_ref, b1_ref, w2_ref, b2_ref,
                 out_ref, outh_ref):
    pp = _unpack8(parts_ref[0], 16) + _unpack8(parts_ref[1], 16)
    t = (jnp.dot(x_ref[...], w1_ref[...], preferred_element_type=jnp.float32)
         + jnp.dot(pp, w1p_ref[...], preferred_element_type=jnp.float32)
         + b1_ref[...])
    h = jnp.maximum(t, 0.0)
    h1 = jnp.maximum(
        jnp.dot(h, w2_ref[...], preferred_element_type=jnp.float32)
        + b2_ref[...], 0.0)
    for c in range(4):
        out_ref[c] = h1[:, c * 16:(c + 1) * 16]
    for hh in range(2):
        outh_ref[hh] = _pack8(
            h1[:, hh * 32:(hh + 1) * 32].astype(jnp.bfloat16), 32)


def _final_kernel(h1c_ref, agg_ref, batch_ref, w3_ref, b3_ref, w4_ref, b4_ref,
                  wl_ref, bl_ref, out_ref, acc_ref):
    i = pl.program_id(0)
    t = jnp.zeros((BN, HID), jnp.float32)
    for c in range(4):
        t = t + jnp.dot(h1c_ref[c], w3_ref[c * 16:(c + 1) * 16, :],
                        preferred_element_type=jnp.float32)
    for hh in range(2):
        t = t + jnp.dot(_unpack8(agg_ref[hh], 32).astype(jnp.float32),
                        w3_ref[hh * 32:(hh + 1) * 32, :],
                        preferred_element_type=jnp.float32)
    h = jnp.maximum(t + b3_ref[...], 0.0)
    h2 = jnp.maximum(
        jnp.dot(h, w4_ref[...], preferred_element_type=jnp.float32)
        + b4_ref[...], 0.0)
    bb = batch_ref[0, 0]
    oh = (bb[:, None] == lax.broadcasted_iota(jnp.int32, (1, NG), 1)
          ).astype(jnp.float32)
    ext = jnp.concatenate([h2, jnp.ones((BN, HID), jnp.float32)], axis=1)
    contrib = lax.dot_general(oh, ext, (((0,), (0,)), ((), ())),
                              preferred_element_type=jnp.float32)

    @pl.when(i == 0)
    def _():
        acc_ref[...] = contrib

    @pl.when(i > 0)
    def _():
        acc_ref[...] = acc_ref[...] + contrib

    @pl.when(i == GRID - 1)
    def _():
        sums = acc_ref[:, :HID]
        cnts = acc_ref[:, HID:]
        pooled = sums / jnp.maximum(cnts, 1.0)
        out_ref[...] = (
            jnp.dot(pooled, wl_ref[...], preferred_element_type=jnp.float32)
            + bl_ref[...])


def kernel(x, edge_index, batch, W1, b1, W2, b2, W3, b3, W4, b4, Wl, bl):
    f32 = jnp.float32
    xp = jnp.pad(x, ((0, NP - N), (0, 16 - x.shape[1])))
    xpad9 = jnp.pad(x, ((0, NP - N), (0, 0)))
    pad_e = E_PAD - E
    srcp = jnp.concatenate(
        [edge_index[0], jnp.zeros((pad_e,), jnp.int32)]).reshape(S_PAD, 128)
    dstp = jnp.concatenate(
        [edge_index[1],
         jnp.full((pad_e,), DUMP, jnp.int32)]).reshape(S_PAD, 128)
    W1p = jnp.pad(W1, ((0, 16 - W1.shape[0]), (0, 0)))
    b1r = b1.reshape(1, HID)
    b2r = b2.reshape(1, HID)
    b3r = b3.reshape(1, HID)
    b4r = b4.reshape(1, HID)
    blr = bl.reshape(1, 32)
    batch2d = jnp.concatenate(
        [batch, jnp.full((NP - N,), NG, jnp.int32)]).reshape(GRID, 1, BN)

    parts = _make_sc_agg1()(xp, srcp, dstp)
    parts2d = parts.reshape(2, NP * 16 // 128, 128)

    _mlp1_out = pl.pallas_call(
        _mlp1_kernel,
        grid=(GRID,),
        in_specs=[
            pl.BlockSpec((BN, 9), lambda i: (i, 0)),
            pl.BlockSpec((2, BN // 8, 128), lambda i: (0, i, 0)),
            pl.BlockSpec((9, HID), lambda i: (0, 0)),
            pl.BlockSpec((16, HID), lambda i: (0, 0)),
            pl.BlockSpec((1, HID), lambda i: (0, 0)),
            pl.BlockSpec((HID, HID), lambda i: (0, 0)),
            pl.BlockSpec((1, HID), lambda i: (0, 0)),
        ],
        out_specs=[pl.BlockSpec((4, BN, 16), lambda i: (0, i, 0)),
                   pl.BlockSpec((2, BN // 8, 256), lambda i: (0, i, 0))],
        out_shape=[jax.ShapeDtypeStruct((4, NP, 16), f32),
                   jax.ShapeDtypeStruct((2, NP * 32 // 256, 256),
                                        jnp.bfloat16)],
    )(xpad9, parts2d, W1, W1p, b1r, W2, b2r)
    h1c, h1h2d = _mlp1_out
    h1h = h1h2d.reshape(2, NP, 32)

    agg = _make_sc_agg2()(h1h, srcp, dstp)
    agg2d = agg.reshape(2, NP * 32 // 256, 256)

    out = pl.pallas_call(
        _final_kernel,
        grid=(GRID,),
        in_specs=[
            pl.BlockSpec((4, BN, 16), lambda i: (0, i, 0)),
            pl.BlockSpec((2, BN // 8, 256), lambda i: (0, i, 0)),
            pl.BlockSpec((1, 1, BN), lambda i: (i, 0, 0)),
            pl.BlockSpec((HID, HID), lambda i: (0, 0)),
            pl.BlockSpec((1, HID), lambda i: (0, 0)),
            pl.BlockSpec((HID, HID), lambda i: (0, 0)),
            pl.BlockSpec((1, HID), lambda i: (0, 0)),
            pl.BlockSpec((HID, 32), lambda i: (0, 0)),
            pl.BlockSpec((1, 32), lambda i: (0, 0)),
        ],
        out_specs=pl.BlockSpec((NG, 32), lambda i: (0, 0)),
        out_shape=jax.ShapeDtypeStruct((NG, 32), f32),
        scratch_shapes=[pltpu.VMEM((NG, 128), f32)],
    )(h1c, agg2d, batch2d, W3, b3r, W4, b4r, Wl, blr)
    return out


# speedup vs baseline: 37.8690x; 1.0578x over previous
import jax
import jax.numpy as jnp
from jax import lax
from jax.experimental import pallas as pl
from jax.experimental.pallas import tpu as pltpu
from jax.experimental.pallas import tpu_sc as plsc

N = 100000
E = 6400000
NG = 256
HID = 64

S_PAD = 50176
E_PAD = S_PAD * 128
ACC_ROWS = 100352
ZROWS = 392
WB = 6272
DUMP = 100351

NP = 102400
BN = 2048
GRID = NP // BN


def _zero_acc(sub, zbuf, acc):
    zero_vec = jnp.zeros(zbuf.shape[1:], zbuf.dtype)

    def zb(i, _):
        zbuf[i, :] = zero_vec
        return 0
    lax.fori_loop(0, ZROWS, zb, 0)
    for z in range(16):
        pltpu.sync_copy(zbuf, acc.at[pl.ds((sub * 16 + z) * ZROWS, ZROWS)])


def _writeback(sub, acc, zbuf, out_slice_fn):
    row0 = sub * WB
    pltpu.sync_copy(acc.at[pl.ds(row0, WB)], out_slice_fn(row0, WB))

    @pl.when(sub == 15)
    def _():
        for k in range(5):
            pltpu.sync_copy(zbuf, out_slice_fn(ACC_ROWS + k * ZROWS, ZROWS))
        rem = NP - ACC_ROWS - 5 * ZROWS
        pltpu.sync_copy(zbuf.at[pl.ds(0, rem)],
                        out_slice_fn(ACC_ROWS + 5 * ZROWS, rem))


def _sigma_slices(ref, q):
    for j in range(4):
        for k in range(8):
            v = ref[q, j, pl.ds(k * 16, 16)]
            o = v & 2047
            ref[q, j, pl.ds(k * 16, 16)] = (
                (v - o) | ((o & 255) << 3) | (o >> 8))


def _edge_sweep(table, src2d, dst2d, acc, sidx, didx, rows,
                isems, gsem, ssems, base, nsup, sig_src, sig_dst):
    nb = nsup // 4

    def idx_issue(slot, sb):
        pltpu.async_copy(src2d.at[pl.ds(sb, 4)], sidx.at[slot], isems[slot])
        pltpu.async_copy(dst2d.at[pl.ds(sb, 4)], didx.at[slot], isems[slot])

    def idx_wait(slot):
        pltpu.make_async_copy(
            src2d.at[pl.ds(0, 4)], sidx.at[slot], isems[slot]).wait()
        pltpu.make_async_copy(
            dst2d.at[pl.ds(0, 4)], didx.at[slot], isems[slot]).wait()

    def scat_drain(r):
        for j in range(4):
            pltpu.make_async_copy(
                rows.at[r, j], acc.at[pl.ds(0, 128)], ssems[r]).wait()

    idx_issue(0, base)
    idx_issue(1, base + 4)

    def body(i, _):
        for q in range(4):
            r = q % 2
            s = i * 4 + q
            if q < 2:
                @pl.when(i > 0)
                def _():
                    scat_drain(r)
            else:
                scat_drain(r)
            tgt = (q + 2) % 4
            if q < 2:
                idx_issue(tgt, base + (s + 2) * 4)
            else:
                @pl.when(i < nb - 1)
                def _():
                    idx_issue(tgt, base + (s + 2) * 4)
            idx_wait(q)
            if sig_src:
                _sigma_slices(sidx, q)
            gcps = [pltpu.async_copy(table.at[sidx.at[q, j]], rows.at[r, j],
                                     gsem) for j in range(4)]
            if sig_dst:
                _sigma_slices(didx, q)
            for c in gcps:
                c.wait()
            for j in range(4):
                pltpu.async_copy(rows.at[r, j], acc.at[didx.at[q, j]],
                                 ssems[r], add=True)
        return 0

    lax.fori_loop(0, nb, body, 0)
    scat_drain(0)
    scat_drain(1)


def _sc_agg1_body(xp, src2d, dst2d, parts, acc, zbuf, sidx, didx,
                  isem0, isem1, isem2, isem3, gsem, ssem0, ssem1):
    core = lax.axis_index("c")
    sub = lax.axis_index("s")
    w = core * 16 + sub
    isems = [isem0, isem1, isem2, isem3]
    ssems = [ssem0, ssem1]

    def inner(rows):
        _zero_acc(sub, zbuf, acc)
        plsc.subcore_barrier()
        _edge_sweep(xp, src2d, dst2d, acc, sidx, didx, rows, isems, gsem,
                    ssems, base=w * (S_PAD // 32), nsup=S_PAD // 32 // 4,
                    sig_src=False, sig_dst=True)
        plsc.subcore_barrier()
        _writeback(sub, acc, zbuf,
                   lambda r0, n: parts.at[core, pl.ds(r0, n)])

    pl.run_scoped(inner, pltpu.VMEM((2, 4, 128, 16), jnp.float32))


def _sc_agg2_body(h1h, src2d, dst2d, agg, acc, zbuf, sidx, didx,
                  isem0, isem1, isem2, isem3, gsem, ssem0, ssem1):
    core = lax.axis_index("c")
    sub = lax.axis_index("s")
    isems = [isem0, isem1, isem2, isem3]
    ssems = [ssem0, ssem1]

    def inner(rows):
        _zero_acc(sub, zbuf, acc)
        plsc.subcore_barrier()
        _edge_sweep(h1h.at[core], src2d, dst2d, acc, sidx, didx, rows,
                    isems, gsem, ssems,
                    base=sub * (S_PAD // 16), nsup=S_PAD // 16 // 4,
                    sig_src=True, sig_dst=True)
        plsc.subcore_barrier()
        _writeback(sub, acc, zbuf,
                   lambda r0, n: agg.at[core, pl.ds(r0, n)])

    pl.run_scoped(inner, pltpu.VMEM((2, 4, 128, 32), jnp.bfloat16))


_SC_SCRATCH1 = [
    pltpu.VMEM_SHARED((ACC_ROWS, 16), jnp.float32),
    pltpu.VMEM((ZROWS, 16), jnp.float32),
    pltpu.VMEM((4, 4, 128), jnp.int32),
    pltpu.VMEM((4, 4, 128), jnp.int32),
] + [pltpu.SemaphoreType.DMA] * 7

_SC_SCRATCH2 = [
    pltpu.VMEM_SHARED((ACC_ROWS, 32), jnp.bfloat16),
    pltpu.VMEM((ZROWS, 32), jnp.bfloat16),
    pltpu.VMEM((4, 4, 128), jnp.int32),
    pltpu.VMEM((4, 4, 128), jnp.int32),
] + [pltpu.SemaphoreType.DMA] * 7


def _make_sc_agg1():
    mesh = plsc.VectorSubcoreMesh(core_axis_name="c", subcore_axis_name="s")
    return pl.kernel(
        _sc_agg1_body,
        out_type=jax.ShapeDtypeStruct((2, NP, 16), jnp.float32),
        mesh=mesh,
        compiler_params=pltpu.CompilerParams(use_tc_tiling_on_sc=False),
        scratch_types=list(_SC_SCRATCH1),
    )


def _make_sc_agg2():
    mesh = plsc.VectorSubcoreMesh(core_axis_name="c", subcore_axis_name="s")
    return pl.kernel(
        _sc_agg2_body,
        out_type=jax.ShapeDtypeStruct((2, NP, 32), jnp.bfloat16),
        mesh=mesh,
        compiler_params=pltpu.CompilerParams(use_tc_tiling_on_sc=False),
        scratch_types=list(_SC_SCRATCH2),
    )


def _unpack8(blk, w):
    return jnp.concatenate([blk[:, g * w:(g + 1) * w] for g in range(8)],
                           axis=0)


def _pack8(nat, w):
    return jnp.concatenate([nat[g * 256:(g + 1) * 256, :] for g in range(8)],
                           axis=1)


def _mlp1_kernel(x_ref, parts_ref, w1_ref, w1p_ref, b1_ref, w2_ref, b2_ref,
                 outh_ref):
    pp = _unpack8(parts_ref[0], 16) + _unpack8(parts_ref[1], 16)
    t = (jnp.dot(x_ref[...], w1_ref[...], preferred_element_type=jnp.float32)
         + jnp.dot(pp, w1p_ref[...], preferred_element_type=jnp.float32)
         + b1_ref[...])
    h = jnp.maximum(t, 0.0)
    h1 = jnp.maximum(
        jnp.dot(h, w2_ref[...], preferred_element_type=jnp.float32)
        + b2_ref[...], 0.0)
    for hh in range(2):
        outh_ref[hh] = _pack8(
            h1[:, hh * 32:(hh + 1) * 32].astype(jnp.bfloat16), 32)


def _final_kernel(h1h_ref, agg_ref, batch_ref, w3_ref, b3_ref, w4_ref, b4_ref,
                  wl_ref, bl_ref, out_ref, acc_ref):
    i = pl.program_id(0)
    t = jnp.zeros((BN, HID), jnp.float32)
    for hh in range(2):
        both = (_unpack8(h1h_ref[hh], 32).astype(jnp.float32)
                + _unpack8(agg_ref[hh], 32).astype(jnp.float32))
        t = t + jnp.dot(both, w3_ref[hh * 32:(hh + 1) * 32, :],
                        preferred_element_type=jnp.float32)
    h = jnp.maximum(t + b3_ref[...], 0.0)
    h2 = jnp.maximum(
        jnp.dot(h, w4_ref[...], preferred_element_type=jnp.float32)
        + b4_ref[...], 0.0)
    bb = batch_ref[0, 0]
    oh = (bb[:, None] == lax.broadcasted_iota(jnp.int32, (1, NG), 1)
          ).astype(jnp.float32)
    ext = jnp.concatenate([h2, jnp.ones((BN, HID), jnp.float32)], axis=1)
    contrib = lax.dot_general(oh, ext, (((0,), (0,)), ((), ())),
                              preferred_element_type=jnp.float32)

    @pl.when(i == 0)
    def _():
        acc_ref[...] = contrib

    @pl.when(i > 0)
    def _():
        acc_ref[...] = acc_ref[...] + contrib

    @pl.when(i == GRID - 1)
    def _():
        sums = acc_ref[:, :HID]
        cnts = acc_ref[:, HID:]
        pooled = sums / jnp.maximum(cnts, 1.0)
        out_ref[...] = (
            jnp.dot(pooled, wl_ref[...], preferred_element_type=jnp.float32)
            + bl_ref[...])


def kernel(x, edge_index, batch, W1, b1, W2, b2, W3, b3, W4, b4, Wl, bl):
    f32 = jnp.float32
    xp = jnp.pad(x, ((0, NP - N), (0, 16 - x.shape[1])))
    xpad9 = jnp.pad(x, ((0, NP - N), (0, 0)))
    pad_e = E_PAD - E
    srcp = jnp.concatenate(
        [edge_index[0], jnp.zeros((pad_e,), jnp.int32)]).reshape(S_PAD, 128)
    dstp = jnp.concatenate(
        [edge_index[1],
         jnp.full((pad_e,), DUMP, jnp.int32)]).reshape(S_PAD, 128)
    W1p = jnp.pad(W1, ((0, 16 - W1.shape[0]), (0, 0)))
    b1r = b1.reshape(1, HID)
    b2r = b2.reshape(1, HID)
    b3r = b3.reshape(1, HID)
    b4r = b4.reshape(1, HID)
    blr = bl.reshape(1, 32)
    batch2d = jnp.concatenate(
        [batch, jnp.full((NP - N,), NG, jnp.int32)]).reshape(GRID, 1, BN)

    parts = _make_sc_agg1()(xp, srcp, dstp)
    parts2d = parts.reshape(2, NP * 16 // 128, 128)

    _mlp1_out = pl.pallas_call(
        _mlp1_kernel,
        grid=(GRID,),
        in_specs=[
            pl.BlockSpec((BN, 9), lambda i: (i, 0)),
            pl.BlockSpec((2, BN // 8, 128), lambda i: (0, i, 0)),
            pl.BlockSpec((9, HID), lambda i: (0, 0)),
            pl.BlockSpec((16, HID), lambda i: (0, 0)),
            pl.BlockSpec((1, HID), lambda i: (0, 0)),
            pl.BlockSpec((HID, HID), lambda i: (0, 0)),
            pl.BlockSpec((1, HID), lambda i: (0, 0)),
        ],
        out_specs=pl.BlockSpec((2, BN // 8, 256), lambda i: (0, i, 0)),
        out_shape=jax.ShapeDtypeStruct((2, NP * 32 // 256, 256),
                                       jnp.bfloat16),
    )(xpad9, parts2d, W1, W1p, b1r, W2, b2r)
    h1h2d = _mlp1_out
    h1h = h1h2d.reshape(2, NP, 32)

    agg = _make_sc_agg2()(h1h, srcp, dstp)
    agg2d = agg.reshape(2, NP * 32 // 256, 256)

    out = pl.pallas_call(
        _final_kernel,
        grid=(GRID,),
        in_specs=[
            pl.BlockSpec((2, BN // 8, 256), lambda i: (0, i, 0)),
            pl.BlockSpec((2, BN // 8, 256), lambda i: (0, i, 0)),
            pl.BlockSpec((1, 1, BN), lambda i: (i, 0, 0)),
            pl.BlockSpec((HID, HID), lambda i: (0, 0)),
            pl.BlockSpec((1, HID), lambda i: (0, 0)),
            pl.BlockSpec((HID, HID), lambda i: (0, 0)),
            pl.BlockSpec((1, HID), lambda i: (0, 0)),
            pl.BlockSpec((HID, 32), lambda i: (0, 0)),
            pl.BlockSpec((1, 32), lambda i: (0, 0)),
        ],
        out_specs=pl.BlockSpec((NG, 32), lambda i: (0, 0)),
        out_shape=jax.ShapeDtypeStruct((NG, 32), f32),
        scratch_shapes=[pltpu.VMEM((NG, 128), f32)],
    )(h1h2d, agg2d, batch2d, W3, b3r, W4, b4r, Wl, blr)
    return out
